# baseline (speedup 1.0000x reference)
import functools

import jax
import jax.numpy as jnp
import numpy as np
from jax.experimental import pallas as pl
from jax.experimental.pallas import tpu as pltpu

K = 1024
CD = 10
D = 64
SCALE = 1.0
INV_TEMP = 100.0
ENT_W = 0.1
COMMIT_W = 0.25
GAMMA = 1.0
B, N = 8, 4096
TOKENS = B * N
TILE = 1024
GRID = TOKENS // TILE

_s = 2.0 * INV_TEMP * SCALE
_CT73 = np.zeros((CD, 136), dtype=np.float32)
for _d in range(7):
    _J = np.arange(128)
    _CT73[_d, :128] = _s * (2.0 * ((_J >> (6 - _d)) & 1) - 1.0)
for _d in range(7, CD):
    _L = np.arange(8)
    _CT73[_d, 128:136] = _s * (2.0 * ((_L >> (9 - _d)) & 1) - 1.0)

_MCOLS = np.zeros((CD, 2), dtype=np.float32)
_MCOLS[:7, 0] = _s
_MCOLS[7:, 1] = _s

_IMASK = (2 ** np.arange(CD - 1, -1, -1)).astype(np.float32)


def _lfq_body(z_ref, wi_ref, bi_ref, wo_ref, bo_ref, ct_ref, mc_ref,
              out_ref, idx_ref, aux_ref,
              avg_acc, sums_acc):
    step = pl.program_id(0)

    @pl.when(step == 0)
    def _init():
        avg_acc[...] = jnp.zeros_like(avg_acc)
        sums_acc[0] = 0.0
        sums_acc[1] = 0.0

    z = z_ref[...]
    x = jax.lax.dot_general(z, wi_ref[...], (((1,), (1,)), ((), ())),
                            preferred_element_type=jnp.float32) + bi_ref[...]
    pos = x > 0
    q = jnp.where(pos, SCALE, -SCALE).astype(jnp.float32)

    y2 = jax.lax.dot_general(q, wo_ref[...], (((1,), (1,)), ((), ())),
                             preferred_element_type=jnp.float32)
    out_ref[...] = y2[:, :D] + bo_ref[...]
    idx_ref[...] = (y2[:, D:D + 1] + (float(K) - 1.0) * 0.5).astype(jnp.int32)

    commit_tile = jnp.sum((x - q) ** 2)

    y = jax.lax.dot_general(x, ct_ref[...], (((1,), (0,)), ((), ())),
                            preferred_element_type=jnp.float32)
    m = jax.lax.dot_general(jnp.abs(x), mc_ref[...], (((1,), (0,)), ((), ())),
                            preferred_element_type=jnp.float32)
    lp7 = y[:, :128] - m[:, 0:1]
    lp3 = y[:, 128:136] - m[:, 1:2]
    e7 = jnp.exp(lp7)
    e3 = jnp.exp(lp3)
    ones1 = jnp.ones((128, 1), dtype=jnp.float32)
    s7 = jax.lax.dot_general(e7, ones1, (((1,), (0,)), ((), ())),
                             preferred_element_type=jnp.float32)
    w7 = jax.lax.dot_general(e7 * lp7, ones1, (((1,), (0,)), ((), ())),
                             preferred_element_type=jnp.float32)
    s3 = jnp.sum(e3, axis=1, keepdims=True)
    w3 = jnp.sum(e3 * lp3, axis=1, keepdims=True)
    r7 = 1.0 / s7
    r3 = 1.0 / s3
    h = jnp.log(s7 * s3) - w7 * r7 - w3 * r3
    ent_tile = jnp.sum(h)

    p3s = e3 * (r7 * r3)
    avg_acc[...] += jax.lax.dot_general(e7, p3s, (((0,), (0,)), ((), ())),
                                        preferred_element_type=jnp.float32)

    sums_acc[0] += ent_tile
    sums_acc[1] += commit_tile

    @pl.when(step == GRID - 1)
    def _fin():
        nt = float(TOKENS)
        pse = sums_acc[0] / nt
        ap = avg_acc[...] / nt
        ce = jnp.sum(-ap * jnp.log(jnp.clip(ap, 1e-20, None)))
        commit = sums_acc[1] / (nt * CD)
        aux = (pse - GAMMA * ce) * ENT_W + COMMIT_W * commit
        aux_ref[...] = jnp.reshape(aux, (1, 1))


@functools.partial(jax.jit, static_argnames=())
def kernel(z_e_x, W_in, b_in, W_out, b_out):
    z2 = z_e_x.reshape(TOKENS, D)
    bi = b_in.reshape(1, CD)
    bo = b_out.reshape(1, D)
    ct = jnp.asarray(_CT73)
    mc = jnp.asarray(_MCOLS)
    wo_aug = jnp.concatenate([W_out, jnp.asarray(_IMASK)[None, :] * 0.5], axis=0)

    out2, idx2, aux = pl.pallas_call(
        _lfq_body,
        grid=(GRID,),
        in_specs=[
            pl.BlockSpec((TILE, D), lambda i: (i, 0)),
            pl.BlockSpec((CD, D), lambda i: (0, 0)),
            pl.BlockSpec((1, CD), lambda i: (0, 0)),
            pl.BlockSpec((D + 1, CD), lambda i: (0, 0)),
            pl.BlockSpec((1, D), lambda i: (0, 0)),
            pl.BlockSpec((CD, 136), lambda i: (0, 0)),
            pl.BlockSpec((CD, 2), lambda i: (0, 0)),
        ],
        out_specs=[
            pl.BlockSpec((TILE, D), lambda i: (i, 0)),
            pl.BlockSpec((TILE, 1), lambda i: (i, 0)),
            pl.BlockSpec((1, 1), lambda i: (0, 0)),
        ],
        out_shape=[
            jax.ShapeDtypeStruct((TOKENS, D), jnp.float32),
            jax.ShapeDtypeStruct((TOKENS, 1), jnp.int32),
            jax.ShapeDtypeStruct((1, 1), jnp.float32),
        ],
        scratch_shapes=[
            pltpu.VMEM((128, 8), jnp.float32),
            pltpu.SMEM((2,), jnp.float32),
        ],
    )(z2, W_in, bi, wo_aug, bo, ct, mc)

    out = out2.reshape(B, N, D)
    indices = idx2.reshape(B, N)
    aux_loss = aux.reshape(())
    return (out, indices, aux_loss)


# speedup vs baseline: 1.5746x; 1.5746x over previous
import functools

import jax
import jax.numpy as jnp
import numpy as np
from jax.experimental import pallas as pl
from jax.experimental.pallas import tpu as pltpu

K = 1024
CD = 10
D = 64
SCALE = 1.0
INV_TEMP = 100.0
ENT_W = 0.1
COMMIT_W = 0.25
GAMMA = 1.0
B, N = 8, 4096
TOKENS = B * N
TILN = 512
GRID = N // TILN
T = B * TILN

_s = 2.0 * INV_TEMP * SCALE
_CT73 = np.zeros((CD, 136), dtype=np.float32)
for _d in range(7):
    _J = np.arange(128)
    _CT73[_d, :128] = _s * (2.0 * ((_J >> (6 - _d)) & 1) - 1.0)
for _d in range(7, CD):
    _L = np.arange(8)
    _CT73[_d, 128:136] = _s * (2.0 * ((_L >> (9 - _d)) & 1) - 1.0)

_MCOLS = np.zeros((CD, 2), dtype=np.float32)
_MCOLS[:7, 0] = _s
_MCOLS[7:, 1] = _s

_IMASK = (2 ** np.arange(CD - 1, -1, -1)).astype(np.float32)


def _lfq_body(z_ref, wi_ref, bi_ref, wo_ref, bo_ref, ct_ref, mc_ref,
              out_ref, idx_ref, aux_ref,
              avg_acc, sums_acc):
    step = pl.program_id(0)

    @pl.when(step == 0)
    def _init():
        avg_acc[...] = jnp.zeros_like(avg_acc)
        sums_acc[0] = 0.0
        sums_acc[1] = 0.0

    z = z_ref[...].reshape(T, D)
    x = jax.lax.dot_general(z, wi_ref[...], (((1,), (1,)), ((), ())),
                            preferred_element_type=jnp.float32) + bi_ref[...]
    pos = x > 0
    q = jnp.where(pos, SCALE, -SCALE).astype(jnp.float32)

    y2 = jax.lax.dot_general(q, wo_ref[...], (((1,), (1,)), ((), ())),
                             preferred_element_type=jnp.float32)
    out_ref[...] = (y2[:, :D] + bo_ref[...]).reshape(B, TILN, D)
    idxf = y2[:, D:D + 1] + (float(K) - 1.0) * 0.5
    idx_ref[...] = idxf.reshape(B, TILN).astype(jnp.int32)

    commit_tile = jnp.sum((x - q) ** 2)

    y = jax.lax.dot_general(x, ct_ref[...], (((1,), (0,)), ((), ())),
                            preferred_element_type=jnp.float32)
    m = jax.lax.dot_general(jnp.abs(x), mc_ref[...], (((1,), (0,)), ((), ())),
                            preferred_element_type=jnp.float32)
    lp7 = y[:, :128] - m[:, 0:1]
    lp3 = y[:, 128:136] - m[:, 1:2]
    e7 = jnp.exp(lp7)
    e3 = jnp.exp(lp3)
    ones1 = jnp.ones((128, 1), dtype=jnp.float32)
    s7 = jax.lax.dot_general(e7, ones1, (((1,), (0,)), ((), ())),
                             preferred_element_type=jnp.float32)
    w7 = jax.lax.dot_general(e7 * lp7, ones1, (((1,), (0,)), ((), ())),
                             preferred_element_type=jnp.float32)
    s3 = jnp.sum(e3, axis=1, keepdims=True)
    w3 = jnp.sum(e3 * lp3, axis=1, keepdims=True)
    r7 = 1.0 / s7
    r3 = 1.0 / s3
    h = jnp.log(s7 * s3) - w7 * r7 - w3 * r3
    ent_tile = jnp.sum(h)

    p3s = e3 * (r7 * r3)
    avg_acc[...] += jax.lax.dot_general(e7, p3s, (((0,), (0,)), ((), ())),
                                        preferred_element_type=jnp.float32)

    sums_acc[0] += ent_tile
    sums_acc[1] += commit_tile

    @pl.when(step == GRID - 1)
    def _fin():
        nt = float(TOKENS)
        pse = sums_acc[0] / nt
        ap = avg_acc[...] / nt
        ce = jnp.sum(-ap * jnp.log(jnp.clip(ap, 1e-20, None)))
        commit = sums_acc[1] / (nt * CD)
        aux = (pse - GAMMA * ce) * ENT_W + COMMIT_W * commit
        aux_ref[...] = jnp.reshape(aux, (1, 1))


@functools.partial(jax.jit, static_argnames=())
def kernel(z_e_x, W_in, b_in, W_out, b_out):
    bi = b_in.reshape(1, CD)
    bo = b_out.reshape(1, D)
    ct = jnp.asarray(_CT73)
    mc = jnp.asarray(_MCOLS)
    wo_aug = jnp.concatenate([W_out, jnp.asarray(_IMASK)[None, :] * 0.5], axis=0)

    out, idx, aux = pl.pallas_call(
        _lfq_body,
        grid=(GRID,),
        in_specs=[
            pl.BlockSpec((B, TILN, D), lambda i: (0, i, 0)),
            pl.BlockSpec((CD, D), lambda i: (0, 0)),
            pl.BlockSpec((1, CD), lambda i: (0, 0)),
            pl.BlockSpec((D + 1, CD), lambda i: (0, 0)),
            pl.BlockSpec((1, D), lambda i: (0, 0)),
            pl.BlockSpec((CD, 136), lambda i: (0, 0)),
            pl.BlockSpec((CD, 2), lambda i: (0, 0)),
        ],
        out_specs=[
            pl.BlockSpec((B, TILN, D), lambda i: (0, i, 0)),
            pl.BlockSpec((B, TILN), lambda i: (0, i)),
            pl.BlockSpec((1, 1), lambda i: (0, 0)),
        ],
        out_shape=[
            jax.ShapeDtypeStruct((B, N, D), jnp.float32),
            jax.ShapeDtypeStruct((B, N), jnp.int32),
            jax.ShapeDtypeStruct((1, 1), jnp.float32),
        ],
        scratch_shapes=[
            pltpu.VMEM((128, 8), jnp.float32),
            pltpu.SMEM((2,), jnp.float32),
        ],
    )(z_e_x, W_in, bi, wo_aug, bo, ct, mc)

    return (out, idx, aux.reshape(()))


# speedup vs baseline: 1.7332x; 1.1007x over previous
import functools

import jax
import jax.numpy as jnp
import numpy as np
from jax.experimental import pallas as pl
from jax.experimental.pallas import tpu as pltpu

K = 1024
CD = 10
D = 64
SCALE = 1.0
INV_TEMP = 100.0
ENT_W = 0.1
COMMIT_W = 0.25
GAMMA = 1.0
B, N = 8, 4096
TOKENS = B * N
TILN = 512
GRID = N // TILN
T = B * TILN

_s = 2.0 * INV_TEMP * SCALE
_CTA = np.zeros((2 * CD, 136), dtype=np.float32)
for _d in range(7):
    _J = np.arange(128)
    _CTA[_d, :128] = _s * (2.0 * ((_J >> (6 - _d)) & 1) - 1.0)
    _CTA[CD + _d, :128] = -_s
for _d in range(7, CD):
    _L = np.arange(8)
    _CTA[_d, 128:136] = _s * (2.0 * ((_L >> (9 - _d)) & 1) - 1.0)
    _CTA[CD + _d, 128:136] = -_s

_IMASK = (2 ** np.arange(CD - 1, -1, -1)).astype(np.float32)


def _lfq_body(z_ref, wi_ref, bi_ref, wo_ref, bo_ref, cta_ref,
              out_ref, idx_ref, aux_ref,
              avg_acc, sums_acc):
    step = pl.program_id(0)

    @pl.when(step == 0)
    def _init():
        avg_acc[...] = jnp.zeros_like(avg_acc)
        sums_acc[0] = 0.0
        sums_acc[1] = 0.0

    z = z_ref[...].reshape(T, D)
    x = jax.lax.dot_general(z, wi_ref[...], (((1,), (1,)), ((), ())),
                            preferred_element_type=jnp.float32) + bi_ref[...]
    pos = x > 0
    q = jnp.where(pos, SCALE, -SCALE).astype(jnp.float32)
    ax = jnp.abs(x)

    y2 = jax.lax.dot_general(q, wo_ref[...], (((1,), (1,)), ((), ())),
                             preferred_element_type=jnp.float32)
    out_ref[...] = (y2[:, :D] + bo_ref[...]).reshape(B, TILN, D)
    idxf = y2[:, D:D + 1] + (float(K) - 1.0) * 0.5
    idx_ref[...] = idxf.reshape(B, TILN).astype(jnp.int32)

    commit_tile = jnp.sum((x - q) ** 2)

    xa = jnp.concatenate([x, ax], axis=1)
    lp = jax.lax.dot_general(xa, cta_ref[...], (((1,), (0,)), ((), ())),
                             preferred_element_type=jnp.float32)
    e7 = jnp.exp(lp[:, :128])
    e3 = jnp.exp(lp[:, 128:136])
    ones1 = jnp.ones((128, 1), dtype=jnp.float32)
    s7 = jax.lax.dot_general(e7, ones1, (((1,), (0,)), ((), ())),
                             preferred_element_type=jnp.float32)
    s3 = jnp.sum(e3, axis=1, keepdims=True)
    p3s = e3 * (1.0 / (s7 * s3))
    avg_acc[...] += jax.lax.dot_general(e7, p3s, (((0,), (0,)), ((), ())),
                                        preferred_element_type=jnp.float32)

    u = (2.0 * _s) * ax
    a = jnp.exp(-u)
    hb = jnp.log1p(a) + u * a / (1.0 + a)
    ent_tile = jnp.sum(hb)

    sums_acc[0] += ent_tile
    sums_acc[1] += commit_tile

    @pl.when(step == GRID - 1)
    def _fin():
        nt = float(TOKENS)
        pse = sums_acc[0] / nt
        ap = avg_acc[...] / nt
        ce = jnp.sum(-ap * jnp.log(jnp.clip(ap, 1e-20, None)))
        commit = sums_acc[1] / (nt * CD)
        aux = (pse - GAMMA * ce) * ENT_W + COMMIT_W * commit
        aux_ref[...] = jnp.reshape(aux, (1, 1))


@functools.partial(jax.jit, static_argnames=())
def kernel(z_e_x, W_in, b_in, W_out, b_out):
    bi = b_in.reshape(1, CD)
    bo = b_out.reshape(1, D)
    cta = jnp.asarray(_CTA)
    wo_aug = jnp.concatenate([W_out, jnp.asarray(_IMASK)[None, :] * 0.5], axis=0)

    out, idx, aux = pl.pallas_call(
        _lfq_body,
        grid=(GRID,),
        in_specs=[
            pl.BlockSpec((B, TILN, D), lambda i: (0, i, 0)),
            pl.BlockSpec((CD, D), lambda i: (0, 0)),
            pl.BlockSpec((1, CD), lambda i: (0, 0)),
            pl.BlockSpec((D + 1, CD), lambda i: (0, 0)),
            pl.BlockSpec((1, D), lambda i: (0, 0)),
            pl.BlockSpec((2 * CD, 136), lambda i: (0, 0)),
        ],
        out_specs=[
            pl.BlockSpec((B, TILN, D), lambda i: (0, i, 0)),
            pl.BlockSpec((B, TILN), lambda i: (0, i)),
            pl.BlockSpec((1, 1), lambda i: (0, 0)),
        ],
        out_shape=[
            jax.ShapeDtypeStruct((B, N, D), jnp.float32),
            jax.ShapeDtypeStruct((B, N), jnp.int32),
            jax.ShapeDtypeStruct((1, 1), jnp.float32),
        ],
        scratch_shapes=[
            pltpu.VMEM((128, 8), jnp.float32),
            pltpu.SMEM((2,), jnp.float32),
        ],
    )(z_e_x, W_in, bi, wo_aug, bo, cta)

    return (out, idx, aux.reshape(()))


# speedup vs baseline: 4.9990x; 2.8843x over previous
import functools

import jax
import jax.numpy as jnp
import numpy as np
from jax.experimental import pallas as pl
from jax.experimental.pallas import tpu as pltpu

K = 1024
CD = 10
D = 64
SCALE = 1.0
INV_TEMP = 100.0
ENT_W = 0.1
COMMIT_W = 0.25
GAMMA = 1.0
B, N = 8, 4096
TOKENS = B * N
TILN = 2048
NCH = N // TILN
GRID = B * NCH

_s = 2.0 * INV_TEMP * SCALE
_CTAT = np.zeros((136, 2 * CD), dtype=np.float32)
for _d in range(7):
    _J = np.arange(128)
    _CTAT[:128, _d] = _s * (2.0 * ((_J >> (6 - _d)) & 1) - 1.0)
    _CTAT[:128, CD + _d] = -_s
for _d in range(7, CD):
    _L = np.arange(8)
    _CTAT[128:136, _d] = _s * (2.0 * ((_L >> (9 - _d)) & 1) - 1.0)
    _CTAT[128:136, CD + _d] = -_s

_IMASK = (2 ** np.arange(CD - 1, -1, -1)).astype(np.float32)


def _lfq_body(z_ref, wi_ref, bi_ref, wo_ref, bo_ref, cta_ref,
              out_ref, idx_ref, aux_ref,
              avg_acc, sums_acc):
    nc = pl.program_id(0)
    bi_idx = pl.program_id(1)

    @pl.when(jnp.logical_and(bi_idx == 0, nc == 0))
    def _init():
        avg_acc[...] = jnp.zeros_like(avg_acc)
        sums_acc[0] = 0.0
        sums_acc[1] = 0.0

    zt = z_ref[...].reshape(D, TILN)
    xt = jax.lax.dot_general(wi_ref[...], zt, (((1,), (0,)), ((), ())),
                             preferred_element_type=jnp.float32) + bi_ref[...]
    pos = xt > 0
    qt = jnp.where(pos, SCALE, -SCALE).astype(jnp.float32)
    axt = jnp.abs(xt)

    y2 = jax.lax.dot_general(wo_ref[...], qt, (((1,), (0,)), ((), ())),
                             preferred_element_type=jnp.float32)
    out_ref[...] = (y2[:D, :] + bo_ref[...]).reshape(1, D, TILN)
    idxf = y2[D:D + 1, :] + (float(K) - 1.0) * 0.5
    idx_ref[pl.ds(bi_idx, 1), :] = idxf.astype(jnp.int32)

    commit_tile = jnp.sum((xt - qt) ** 2)

    xa = jnp.concatenate([xt, axt], axis=0)
    lp = jax.lax.dot_general(cta_ref[...], xa, (((1,), (0,)), ((), ())),
                             preferred_element_type=jnp.float32)
    e7 = jnp.exp(lp[:128, :])
    e3 = jnp.exp(lp[128:136, :])
    s7 = jnp.sum(e7, axis=0, keepdims=True)
    s3 = jnp.sum(e3, axis=0, keepdims=True)
    s73 = s7 * s3
    p3s = e3 * (1.0 / s73)
    p3t = jnp.swapaxes(p3s, 0, 1)
    avg_acc[...] += jax.lax.dot_general(e7, p3t, (((1,), (0,)), ((), ())),
                                        preferred_element_type=jnp.float32)

    u = (2.0 * _s) * axt
    a = jnp.exp(-u)
    g = u * a / (1.0 + a)
    ent_tile = jnp.sum(g) + jnp.sum(jnp.log(s73))

    sums_acc[0] += ent_tile
    sums_acc[1] += commit_tile

    @pl.when(jnp.logical_and(bi_idx == B - 1, nc == NCH - 1))
    def _fin():
        nt = float(TOKENS)
        pse = sums_acc[0] / nt
        ap = avg_acc[...] / nt
        ce = jnp.sum(-ap * jnp.log(jnp.clip(ap, 1e-20, None)))
        commit = sums_acc[1] / (nt * CD)
        aux = (pse - GAMMA * ce) * ENT_W + COMMIT_W * commit
        aux_ref[...] = jnp.reshape(aux, (1, 1))


@functools.partial(jax.jit, static_argnames=())
def kernel(z_e_x, W_in, b_in, W_out, b_out):
    zt = jnp.swapaxes(z_e_x, 1, 2)
    bi = b_in.reshape(CD, 1)
    bo = b_out.reshape(D, 1)
    cta = jnp.asarray(_CTAT)
    wo_aug = jnp.concatenate([W_out, jnp.asarray(_IMASK)[None, :] * 0.5], axis=0)

    out_t, idx, aux = pl.pallas_call(
        _lfq_body,
        grid=(NCH, B),
        in_specs=[
            pl.BlockSpec((1, D, TILN), lambda n, b: (b, 0, n)),
            pl.BlockSpec((CD, D), lambda n, b: (0, 0)),
            pl.BlockSpec((CD, 1), lambda n, b: (0, 0)),
            pl.BlockSpec((D + 1, CD), lambda n, b: (0, 0)),
            pl.BlockSpec((D, 1), lambda n, b: (0, 0)),
            pl.BlockSpec((136, 2 * CD), lambda n, b: (0, 0)),
        ],
        out_specs=[
            pl.BlockSpec((1, D, TILN), lambda n, b: (b, 0, n)),
            pl.BlockSpec((8, TILN), lambda n, b: (0, n)),
            pl.BlockSpec((1, 1), lambda n, b: (0, 0)),
        ],
        out_shape=[
            jax.ShapeDtypeStruct((B, D, N), jnp.float32),
            jax.ShapeDtypeStruct((B, N), jnp.int32),
            jax.ShapeDtypeStruct((1, 1), jnp.float32),
        ],
        scratch_shapes=[
            pltpu.VMEM((128, 8), jnp.float32),
            pltpu.SMEM((2,), jnp.float32),
        ],
    )(zt, W_in, bi, wo_aug, bo, cta)

    out = jnp.swapaxes(out_t, 1, 2)
    return (out, idx, aux.reshape(()))


# speedup vs baseline: 5.9281x; 1.1858x over previous
import functools

import jax
import jax.numpy as jnp
import numpy as np
from jax.experimental import pallas as pl
from jax.experimental.pallas import tpu as pltpu

K = 1024
CD = 10
D = 64
SCALE = 1.0
INV_TEMP = 100.0
ENT_W = 0.1
COMMIT_W = 0.25
GAMMA = 1.0
B, N = 8, 4096
TOKENS = B * N
NSPLIT = 2
CHN = N // NSPLIT

_s = 2.0 * INV_TEMP * SCALE
_CTAT = np.zeros((136, 2 * CD), dtype=np.float32)
for _d in range(7):
    _J = np.arange(128)
    _CTAT[:128, _d] = _s * (2.0 * ((_J >> (6 - _d)) & 1) - 1.0)
    _CTAT[:128, CD + _d] = -_s
for _d in range(7, CD):
    _L = np.arange(8)
    _CTAT[128:136, _d] = _s * (2.0 * ((_L >> (9 - _d)) & 1) - 1.0)
    _CTAT[128:136, CD + _d] = -_s

_IMASK = (2 ** np.arange(CD - 1, -1, -1)).astype(np.float32)


def _lfq_body(z_ref, wi_ref, bi_ref, wo_ref, bo_ref, cta_ref,
              out_ref, idx_ref, aux_ref,
              avg_acc, sums_acc):
    b = pl.program_id(0)

    @pl.when(b == 0)
    def _init():
        avg_acc[...] = jnp.zeros_like(avg_acc)
        sums_acc[0] = 0.0
        sums_acc[1] = 0.0

    ent_tile = 0.0
    commit_tile = 0.0
    avg_upd = jnp.zeros((128, 8), dtype=jnp.float32)
    for h in range(NSPLIT):
        sl = slice(h * CHN, (h + 1) * CHN)
        zt = z_ref[0, :, sl]
        xt = jax.lax.dot_general(wi_ref[...], zt, (((1,), (0,)), ((), ())),
                                 preferred_element_type=jnp.float32) + bi_ref[...]
        qt = jnp.where(xt > 0, SCALE, -SCALE).astype(jnp.float32)
        axt = jnp.abs(xt)

        y2 = jax.lax.dot_general(wo_ref[...], qt, (((1,), (0,)), ((), ())),
                                 preferred_element_type=jnp.float32)
        out_ref[0, :, sl] = y2[:D, :] + bo_ref[...]
        idxf = y2[D:D + 1, :] + (float(K) - 1.0) * 0.5
        idx_ref[pl.ds(b, 1), sl] = idxf.astype(jnp.int32)

        commit_tile += jnp.sum((xt - qt) ** 2)

        xa = jnp.concatenate([xt, axt], axis=0)
        lp = jax.lax.dot_general(cta_ref[...], xa, (((1,), (0,)), ((), ())),
                                 preferred_element_type=jnp.float32)
        e7 = jnp.exp(lp[:128, :])
        e3 = jnp.exp(lp[128:136, :])
        s7 = jnp.sum(e7, axis=0, keepdims=True)
        s3 = jnp.sum(e3, axis=0, keepdims=True)
        s73 = s7 * s3
        p3s = e3 * (1.0 / s73)
        p3t = jnp.swapaxes(p3s, 0, 1)
        avg_upd += jax.lax.dot_general(e7, p3t, (((1,), (0,)), ((), ())),
                                       preferred_element_type=jnp.float32)

        u = (2.0 * _s) * axt
        a = jnp.exp(-u)
        g = u * a / (1.0 + a)
        ent_tile += jnp.sum(g) + jnp.sum(jnp.log(s73))

    avg_acc[...] += avg_upd
    sums_acc[0] += ent_tile
    sums_acc[1] += commit_tile

    @pl.when(b == B - 1)
    def _fin():
        nt = float(TOKENS)
        pse = sums_acc[0] / nt
        ap = avg_acc[...] / nt
        ce = jnp.sum(-ap * jnp.log(jnp.clip(ap, 1e-20, None)))
        commit = sums_acc[1] / (nt * CD)
        aux = (pse - GAMMA * ce) * ENT_W + COMMIT_W * commit
        aux_ref[...] = jnp.reshape(aux, (1, 1))


@functools.partial(jax.jit, static_argnames=())
def kernel(z_e_x, W_in, b_in, W_out, b_out):
    zt = jnp.swapaxes(z_e_x, 1, 2)
    bi = b_in.reshape(CD, 1)
    bo = b_out.reshape(D, 1)
    cta = jnp.asarray(_CTAT)
    wo_aug = jnp.concatenate([W_out, jnp.asarray(_IMASK)[None, :] * 0.5], axis=0)

    out_t, idx, aux = pl.pallas_call(
        _lfq_body,
        grid=(B,),
        in_specs=[
            pl.BlockSpec((1, D, N), lambda b: (b, 0, 0)),
            pl.BlockSpec((CD, D), lambda b: (0, 0)),
            pl.BlockSpec((CD, 1), lambda b: (0, 0)),
            pl.BlockSpec((D + 1, CD), lambda b: (0, 0)),
            pl.BlockSpec((D, 1), lambda b: (0, 0)),
            pl.BlockSpec((136, 2 * CD), lambda b: (0, 0)),
        ],
        out_specs=[
            pl.BlockSpec((1, D, N), lambda b: (b, 0, 0)),
            pl.BlockSpec((B, N), lambda b: (0, 0)),
            pl.BlockSpec((1, 1), lambda b: (0, 0)),
        ],
        out_shape=[
            jax.ShapeDtypeStruct((B, D, N), jnp.float32),
            jax.ShapeDtypeStruct((B, N), jnp.int32),
            jax.ShapeDtypeStruct((1, 1), jnp.float32),
        ],
        scratch_shapes=[
            pltpu.VMEM((128, 8), jnp.float32),
            pltpu.SMEM((2,), jnp.float32),
        ],
    )(zt, W_in, bi, wo_aug, bo, cta)

    out = jnp.swapaxes(out_t, 1, 2)
    return (out, idx, aux.reshape(()))
